# baseline (speedup 1.0000x reference)
import functools

import jax
import jax.numpy as jnp
from jax import lax
from jax.experimental import pallas as pl
from jax.experimental.pallas import tpu as pltpu
from jax.experimental.pallas import tpu_sc as plsc

N_NODES = 10000
D_FEAT = 128
NUM_CLUSTERS = 5000

NC = 2
NS = 16

NODES_PAD = 10240
NPT = NODES_PAD // NS
CHUNK = 80
NCHUNK = NPT // CHUNK
XCLAMP = N_NODES - CHUNK

SEGS_CORE = 2560
SEG_PAD_CORE = 2688
DIV_PER_TILE = SEGS_CORE // NS
DUMP = SEGS_CORE

_mesh = plsc.VectorSubcoreMesh(core_axis_name="c", subcore_axis_name="s")


@functools.partial(
    pl.kernel,
    out_type=jax.ShapeDtypeStruct((NC * SEGS_CORE, D_FEAT), jnp.float32),
    mesh=_mesh,
    scratch_types=[
        pltpu.VMEM((2, CHUNK, D_FEAT), jnp.float32),
        pltpu.VMEM((NPT,), jnp.int32),
        pltpu.VMEM((NCHUNK, CHUNK), jnp.int32),
        pltpu.VMEM((CHUNK, D_FEAT), jnp.float32),
        pltpu.VMEM((CHUNK, D_FEAT), jnp.float32),
        pltpu.VMEM((CHUNK, D_FEAT), jnp.float32),
        pltpu.VMEM((CHUNK, D_FEAT), jnp.float32),
        pltpu.VMEM_SHARED((SEG_PAD_CORE, D_FEAT), jnp.float32),
        pltpu.VMEM_SHARED((SEG_PAD_CORE, D_FEAT), jnp.float32),
        pltpu.SemaphoreType.DMA,
        pltpu.SemaphoreType.DMA,
    ],
)
def _sc_pool(x_hbm, cid_hbm, out_hbm,
             rows_v, cid_v, cid2_v, ones_v, z_v, sv, cv,
             sums_sh, cnts_sh, sem0, sem1):
    c = lax.axis_index("c")
    s = lax.axis_index("s")
    base = s * NPT

    sems = [sem0, sem1]
    copies = [
        pltpu.make_async_copy(
            x_hbm.at[pl.ds(jnp.minimum(base + j * CHUNK, XCLAMP), CHUNK)],
            rows_v.at[j % 2], sems[j % 2])
        for j in range(NCHUNK)
    ]
    copies[0].start()
    cb = jnp.minimum(base, N_NODES - NPT)
    shift = base - cb
    pltpu.sync_copy(cid_hbm.at[pl.ds(cb, NPT)], cid_v)

    lo = c * SEGS_CORE
    dump16 = jnp.full((16,), DUMP, jnp.int32)
    iota16 = lax.iota(jnp.int32, 16)
    for r in range(NCHUNK):
        for k in range(CHUNK // 16):
            p0 = r * CHUNK + k * 16
            off = jnp.minimum(shift + p0, NPT - 16)
            ids = cid_v[pl.ds(off, 16)]
            valid = (base + p0 + iota16) < N_NODES
            loc = ids - lo
            ok = valid & (loc >= 0) & (loc < SEGS_CORE)
            cid2_v[r, pl.ds(k * 16, 16)] = jnp.where(ok, loc, dump16)

    zeros16 = jnp.zeros((16,), jnp.float32)
    ones16 = jnp.ones((16,), jnp.float32)
    for i in range(CHUNK):
        for k in range(D_FEAT // 16):
            z_v[i, pl.ds(k * 16, 16)] = zeros16
            ones_v[i, pl.ds(k * 16, 16)] = ones16

    zb = s * DIV_PER_TILE
    for i in range(DIV_PER_TILE // CHUNK):
        pltpu.sync_copy(z_v, sums_sh.at[pl.ds(zb + i * CHUNK, CHUNK)])
        pltpu.sync_copy(z_v, cnts_sh.at[pl.ds(zb + i * CHUNK, CHUNK)])

    plsc.subcore_barrier()

    for j in range(NCHUNK):
        copies[j].wait()
        if j + 1 < NCHUNK:
            copies[j + 1].start()
        pltpu.sync_copy(rows_v.at[j % 2], sums_sh.at[cid2_v.at[j]], add=True)
        pltpu.sync_copy(ones_v, cnts_sh.at[cid2_v.at[j]], add=True)

    plsc.subcore_barrier()

    for rnd in range(DIV_PER_TILE // CHUNK):
        r0 = zb + rnd * CHUNK
        pltpu.sync_copy(sums_sh.at[pl.ds(r0, CHUNK)], sv)
        pltpu.sync_copy(cnts_sh.at[pl.ds(r0, CHUNK)], cv)
        for j in range(CHUNK):
            cnt = cv[j, pl.ds(0, 16)]
            rec = ones16 / jnp.maximum(cnt, ones16)
            for k in range(D_FEAT // 16):
                sv[j, pl.ds(k * 16, 16)] = sv[j, pl.ds(k * 16, 16)] * rec
        pltpu.sync_copy(sv, out_hbm.at[pl.ds(c * SEGS_CORE + r0, CHUNK)])


def kernel(x, edge_index, cluster):
    del edge_index
    return _sc_pool(x, cluster)[:NUM_CLUSTERS]


# speedup vs baseline: 1.0180x; 1.0180x over previous
import functools

import jax
import jax.numpy as jnp
from jax import lax
from jax.experimental import pallas as pl
from jax.experimental.pallas import tpu as pltpu
from jax.experimental.pallas import tpu_sc as plsc

N_NODES = 10000
D_FEAT = 128
NUM_CLUSTERS = 5000

NC = 2
NS = 16

NODES_PAD = 10240
NPT = NODES_PAD // NS
CHUNK = 80
NCHUNK = NPT // CHUNK
XCLAMP = N_NODES - CHUNK

SEGS_CORE = 2560
SEG_PAD_CORE = 2688
DIV_PER_TILE = SEGS_CORE // NS
DUMP = SEGS_CORE

_mesh = plsc.VectorSubcoreMesh(core_axis_name="c", subcore_axis_name="s")


@functools.partial(
    pl.kernel,
    out_type=jax.ShapeDtypeStruct((NC * SEGS_CORE, D_FEAT), jnp.float32),
    mesh=_mesh,
    scratch_types=[
        pltpu.VMEM((2, CHUNK, D_FEAT), jnp.float32),
        pltpu.VMEM((NCHUNK, CHUNK), jnp.int32),
        pltpu.VMEM((NCHUNK, CHUNK), jnp.int32),
        pltpu.VMEM((CHUNK, D_FEAT), jnp.float32),
        pltpu.VMEM((CHUNK, D_FEAT), jnp.float32),
        pltpu.VMEM((CHUNK, D_FEAT), jnp.float32),
        pltpu.VMEM((CHUNK, D_FEAT), jnp.float32),
        pltpu.VMEM_SHARED((SEG_PAD_CORE, D_FEAT), jnp.float32),
        pltpu.VMEM_SHARED((SEG_PAD_CORE, D_FEAT), jnp.float32),
        pltpu.SemaphoreType.DMA,
        pltpu.SemaphoreType.DMA,
    ],
)
def _sc_pool(x_hbm, cid_hbm, out_hbm,
             rows_v, cid_v, cid2_v, ones_v, z_v, sv, cv,
             sums_sh, cnts_sh, sem0, sem1):
    c = lax.axis_index("c")
    s = lax.axis_index("s")
    base = s * NPT

    sems = [sem0, sem1]
    copies = [
        pltpu.make_async_copy(
            x_hbm.at[pl.ds(jnp.minimum(base + j * CHUNK, XCLAMP), CHUNK)],
            rows_v.at[j % 2], sems[j % 2])
        for j in range(NCHUNK)
    ]
    copies[0].start()
    pltpu.sync_copy(cid_hbm.at[pl.ds(s * NCHUNK, NCHUNK)], cid_v)

    lo = c * SEGS_CORE
    dump16 = jnp.full((16,), DUMP, jnp.int32)
    for r in range(NCHUNK):
        for k in range(CHUNK // 16):
            ids = cid_v[r, pl.ds(k * 16, 16)]
            loc = ids - lo
            ok = (loc >= 0) & (loc < SEGS_CORE)
            cid2_v[r, pl.ds(k * 16, 16)] = jnp.where(ok, loc, dump16)

    zeros16 = jnp.zeros((16,), jnp.float32)
    ones16 = jnp.ones((16,), jnp.float32)
    for i in range(CHUNK):
        for k in range(D_FEAT // 16):
            z_v[i, pl.ds(k * 16, 16)] = zeros16
            ones_v[i, pl.ds(k * 16, 16)] = ones16

    zb = s * DIV_PER_TILE
    for i in range(DIV_PER_TILE // CHUNK):
        pltpu.sync_copy(z_v, sums_sh.at[pl.ds(zb + i * CHUNK, CHUNK)])
        pltpu.sync_copy(z_v, cnts_sh.at[pl.ds(zb + i * CHUNK, CHUNK)])

    plsc.subcore_barrier()

    for j in range(NCHUNK):
        copies[j].wait()
        if j + 1 < NCHUNK:
            copies[j + 1].start()
        pltpu.sync_copy(rows_v.at[j % 2], sums_sh.at[cid2_v.at[j]], add=True)
        pltpu.sync_copy(ones_v, cnts_sh.at[cid2_v.at[j]], add=True)

    plsc.subcore_barrier()

    for rnd in range(DIV_PER_TILE // CHUNK):
        r0 = zb + rnd * CHUNK
        pltpu.sync_copy(sums_sh.at[pl.ds(r0, CHUNK)], sv)
        pltpu.sync_copy(cnts_sh.at[pl.ds(r0, CHUNK)], cv)
        for j in range(CHUNK):
            cnt = cv[j, pl.ds(0, 16)]
            rec = ones16 / jnp.maximum(cnt, ones16)
            for k in range(D_FEAT // 16):
                sv[j, pl.ds(k * 16, 16)] = sv[j, pl.ds(k * 16, 16)] * rec
        pltpu.sync_copy(sv, out_hbm.at[pl.ds(c * SEGS_CORE + r0, CHUNK)])


def kernel(x, edge_index, cluster):
    del edge_index
    cid_pad = jnp.full((NODES_PAD,), NUM_CLUSTERS, jnp.int32).at[:N_NODES].set(cluster)
    cid_r = cid_pad.reshape(NS * NCHUNK, CHUNK)
    return _sc_pool(x, cid_r)[:NUM_CLUSTERS]
